# Initial kernel scaffold; baseline (speedup 1.0000x reference)
import jax
import jax.numpy as jnp
from jax.experimental import pallas as pl


def kernel(z, dictionary, usage_ema):
    raise NotImplementedError("write your pallas kernel here")


# speedup vs baseline: 133.6490x; 133.6490x over previous
import functools

import jax
import jax.numpy as jnp
from jax.experimental import pallas as pl

NUM_EMBEDDINGS = 1024
EMBEDDING_DIM = 64
SPARSITY = 5
PATCH = 2
COMMIT = 0.25
EPS = 1e-10
ALPHA = 0.3
ATOM_DIM = EMBEDDING_DIM * PATCH * PATCH


def _fsub(L, b):
    k = len(b)
    y = []
    for i in range(k):
        acc = b[i]
        for j in range(i):
            acc = acc - L[i][j] * y[j]
        y.append(acc / L[i][i])
    return y


def _bsub(L, y):
    k = len(y)
    g = [None] * k
    for i in reversed(range(k)):
        acc = y[i]
        for j in range(i + 1, k):
            acc = acc - L[j][i] * g[j]
        g[i] = acc / L[i][i]
    return g


def _omp_kernel(x_ref, d_ref, boost_ref, recon_ref, err_ref):
    Bb = x_ref.shape[0]
    N = d_ref.shape[1]
    X = x_ref[...]
    D = d_ref[...]
    norms = jnp.maximum(jnp.sqrt(jnp.sum(D * D, axis=0, keepdims=True)), EPS)
    Dn = D / norms
    boost = boost_ref[...]

    f32 = jnp.float32
    h_bar = jax.lax.dot_general(X, Dn, (((1,), (0,)), ((), ())),
                                preferred_element_type=f32)
    iota_n = jax.lax.broadcasted_iota(jnp.int32, (Bb, N), 1)

    h = h_bar
    selected = jnp.zeros((Bb, N), f32)
    dcols = []
    hst = []
    L = []
    gamma = None
    recon = None
    for k in range(1, SPARSITY + 1):
        scores = jnp.abs(h) * (1.0 - selected) * boost
        m = jnp.max(scores, axis=1, keepdims=True)
        idx = jnp.min(jnp.where(scores == m, iota_n, N), axis=1, keepdims=True)
        onehot = (iota_n == idx).astype(f32)
        selected = jnp.maximum(selected, onehot)
        d_new = jax.lax.dot_general(onehot, Dn, (((1,), (1,)), ((), ())),
                                    preferred_element_type=f32)
        if k > 1:
            g_col = [jnp.sum(dcols[j] * d_new, axis=1, keepdims=True)
                     for j in range(k - 1)]
            w = _fsub(L, g_col)
            wsq = w[0] * w[0]
            for j in range(1, k - 1):
                wsq = wsq + w[j] * w[j]
            wc = jnp.sqrt(jnp.maximum(1.0 - wsq, 1e-12))
            L.append(w + [wc])
        else:
            L.append([jnp.ones((Bb, 1), f32)])
        dcols.append(d_new)
        hst.append(jnp.sum(h_bar * onehot, axis=1, keepdims=True))
        y = _fsub(L, hst)
        gamma = _bsub(L, y)
        recon = gamma[0] * dcols[0]
        for j in range(1, k):
            recon = recon + gamma[j] * dcols[j]
        if k < SPARSITY:
            beta = jax.lax.dot_general(recon, Dn, (((1,), (0,)), ((), ())),
                                       preferred_element_type=f32)
            h = h_bar - beta
    recon_ref[...] = recon
    diff = recon - X
    blk_err = jnp.sum(diff * diff, keepdims=True)

    @pl.when(pl.program_id(0) == 0)
    def _init():
        err_ref[...] = jnp.zeros_like(err_ref)

    err_ref[...] += blk_err


def _run_omp(patches, dictionary, boost, block_b, interpret=False):
    T, M = patches.shape
    N = dictionary.shape[1]
    grid = (T // block_b,)
    recon, err = pl.pallas_call(
        _omp_kernel,
        grid=grid,
        in_specs=[
            pl.BlockSpec((block_b, M), lambda i: (i, 0)),
            pl.BlockSpec((M, N), lambda i: (0, 0)),
            pl.BlockSpec((1, N), lambda i: (0, 0)),
        ],
        out_specs=[
            pl.BlockSpec((block_b, M), lambda i: (i, 0)),
            pl.BlockSpec((1, 1), lambda i: (0, 0)),
        ],
        out_shape=[
            jax.ShapeDtypeStruct((T, M), jnp.float32),
            jax.ShapeDtypeStruct((1, 1), jnp.float32),
        ],
        interpret=interpret,
    )(patches, dictionary, boost)
    return recon, err


@functools.partial(jax.jit, static_argnames=("interpret", "block_b"))
def _dict_forward(z, dictionary, usage_ema, interpret=False, block_b=1024):
    Bz, C, H, W = z.shape
    P = PATCH
    Hp, Wp = H // P, W // P
    patches = z.reshape(Bz, C, Hp, P, Wp, P).transpose(0, 2, 4, 1, 3, 5)
    patches = patches.reshape(Bz * Hp * Wp, C * P * P)
    N = dictionary.shape[1]
    usage = usage_ema / jnp.maximum(usage_ema.sum(), EPS)
    uniform = 1.0 / max(1.0, float(N))
    boost = jnp.minimum((uniform / jnp.maximum(usage, EPS)) ** ALPHA, 8.0)
    recon, err = _run_omp(patches, dictionary, boost.reshape(1, N),
                          block_b, interpret=interpret)
    loss = (1.0 + COMMIT) * err[0, 0] / (patches.shape[0] * patches.shape[1])
    zq = recon.reshape(Bz, Hp, Wp, C, P, P).transpose(0, 3, 1, 4, 2, 5)
    zq = zq.reshape(Bz, C, H, W)
    return zq, loss


def kernel(z, dictionary, usage_ema):
    return _dict_forward(z, dictionary, usage_ema)
